# baseline (speedup 1.0000x reference)
import jax
import jax.numpy as jnp
from jax import lax
from jax.experimental import pallas as pl
from jax.experimental.pallas import tpu as pltpu
from jax.experimental.pallas import tpu_sc as plsc

_NW = 32
_LANES = 16


def _sc_body(grid_hbm, coords_hbm, agent_hbm, out_hbm,
             coords_v, agent_v, rows_v, sem):
    s_sz, ch_sz, h, w_sz = grid_hbm.shape
    rows_per_tile = (s_sz * ch_sz) // _NW
    tiles_per_scene = ch_sz // rows_per_tile

    wid = lax.axis_index("s") * 2 + lax.axis_index("c")
    s = wid // tiles_per_scene
    ch0 = pl.multiple_of((wid % tiles_per_scene) * rows_per_tile,
                         rows_per_tile)

    pltpu.sync_copy(grid_hbm.at[s, pl.ds(ch0, rows_per_tile)],
                    out_hbm.at[s, pl.ds(ch0, rows_per_tile)])

    pltpu.sync_copy(coords_hbm, coords_v)
    cv = coords_v[...]
    c0 = cv[0]
    c1 = cv[1]
    sid = cv[2]

    @pl.when(s == sid)
    def _():
        c1a = pl.multiple_of((c1 // _LANES) * _LANES, _LANES)
        lc = c1 - c1a
        pltpu.sync_copy(agent_hbm.at[pl.ds(ch0, rows_per_tile)], agent_v)
        gathers = [
            pltpu.async_copy(
                out_hbm.at[s, ch0 + i, c0, pl.ds(c1a, _LANES)],
                rows_v.at[i], sem)
            for i in range(rows_per_tile)
        ]
        for g in gathers:
            g.wait()
        av = agent_v[...]
        iota = lax.broadcasted_iota(jnp.int32, (_LANES,), 0)
        mask = iota == lc
        for i in range(rows_per_tile):
            v = rows_v[i]
            rows_v[i] = jnp.where(mask, jnp.maximum(v, av[i]), v)
        scatters = [
            pltpu.async_copy(
                rows_v.at[i],
                out_hbm.at[s, ch0 + i, c0, pl.ds(c1a, _LANES)], sem)
            for i in range(rows_per_tile)
        ]
        for sc in scatters:
            sc.wait()


def kernel(input_grid, input_state_of_agent, coordinates_at_last_frame, scene_id):
    s, ch, h, w = input_grid.shape
    coords = jnp.zeros((_LANES,), jnp.int32)
    coords = coords.at[0].set(coordinates_at_last_frame[0].astype(jnp.int32))
    coords = coords.at[1].set(coordinates_at_last_frame[1].astype(jnp.int32))
    coords = coords.at[2].set(jnp.asarray(scene_id, jnp.int32))
    agent = input_state_of_agent.reshape(ch).astype(jnp.float32)

    sc_kernel = pl.kernel(
        _sc_body,
        out_type=jax.ShapeDtypeStruct((s, ch, h, w), jnp.float32),
        mesh=plsc.VectorSubcoreMesh(
            core_axis_name="c", subcore_axis_name="s",
            num_cores=2, num_subcores=16,
        ),
        scratch_types=[
            pltpu.VMEM((_LANES,), jnp.int32),
            pltpu.VMEM((_LANES,), jnp.float32),
            pltpu.VMEM((_LANES, _LANES), jnp.float32),
            pltpu.SemaphoreType.DMA,
        ],
    )
    return sc_kernel(input_grid, coords, agent)


# speedup vs baseline: 39.5832x; 39.5832x over previous
import jax
import jax.numpy as jnp
from jax import lax
from jax.experimental import pallas as pl
from jax.experimental.pallas import tpu as pltpu
from jax.experimental.pallas import tpu_sc as plsc

_NW = 32
_LANES = 16


def _sc_body(coords_hbm, agent_hbm, out_hbm,
             coords_v, agent_v, rows_v, sem):
    s_sz, ch_sz, h, w_sz = out_hbm.shape
    rows_per_tile = (s_sz * ch_sz) // _NW
    tiles_per_scene = ch_sz // rows_per_tile

    wid = lax.axis_index("s") * 2 + lax.axis_index("c")
    s = wid // tiles_per_scene
    ch0 = pl.multiple_of((wid % tiles_per_scene) * rows_per_tile,
                         rows_per_tile)

    pltpu.sync_copy(coords_hbm, coords_v)
    cv = coords_v[...]
    c0 = cv[0]
    c1 = cv[1]
    sid = cv[2]

    @pl.when(s == sid)
    def _():
        c1a = pl.multiple_of((c1 // _LANES) * _LANES, _LANES)
        lc = c1 - c1a
        pltpu.sync_copy(agent_hbm.at[pl.ds(ch0, rows_per_tile)], agent_v)
        gathers = [
            pltpu.async_copy(
                out_hbm.at[s, ch0 + i, c0, pl.ds(c1a, _LANES)],
                rows_v.at[i], sem)
            for i in range(rows_per_tile)
        ]
        for g in gathers:
            g.wait()
        av = agent_v[...]
        iota = lax.broadcasted_iota(jnp.int32, (_LANES,), 0)
        mask = iota == lc
        for i in range(rows_per_tile):
            v = rows_v[i]
            rows_v[i] = jnp.where(mask, jnp.maximum(v, av[i]), v)
        scatters = [
            pltpu.async_copy(
                rows_v.at[i],
                out_hbm.at[s, ch0 + i, c0, pl.ds(c1a, _LANES)], sem)
            for i in range(rows_per_tile)
        ]
        for sc in scatters:
            sc.wait()


def kernel(input_grid, input_state_of_agent, coordinates_at_last_frame, scene_id):
    s, ch, h, w = input_grid.shape
    coords = jnp.zeros((_LANES,), jnp.int32)
    coords = coords.at[0].set(coordinates_at_last_frame[0].astype(jnp.int32))
    coords = coords.at[1].set(coordinates_at_last_frame[1].astype(jnp.int32))
    coords = coords.at[2].set(jnp.asarray(scene_id, jnp.int32))
    agent = input_state_of_agent.reshape(ch).astype(jnp.float32)

    grid_ref = jax.new_ref(input_grid)

    sc_kernel = pl.kernel(
        _sc_body,
        out_type=(),
        mesh=plsc.VectorSubcoreMesh(
            core_axis_name="c", subcore_axis_name="s",
            num_cores=2, num_subcores=16,
        ),
        scratch_types=[
            pltpu.VMEM((_LANES,), jnp.int32),
            pltpu.VMEM((_LANES,), jnp.float32),
            pltpu.VMEM((_LANES, _LANES), jnp.float32),
            pltpu.SemaphoreType.DMA,
        ],
    )
    sc_kernel(coords, agent, grid_ref)
    return jax.freeze(grid_ref)
